# baseline (speedup 1.0000x reference)
import jax
import jax.numpy as jnp
from jax.experimental import pallas as pl
from jax.experimental.pallas import tpu as pltpu

_BM = 80


def _gcn_body(x_ref, adj_ref, wt_ref, b_ref, out_ref, support_ref, acc_ref):
    i = pl.program_id(0)

    @pl.when(i == 0)
    def _():
        support_ref[...] = jax.lax.dot_general(
            x_ref[...],
            wt_ref[...],
            (((1,), (1,)), ((), ())),
            preferred_element_type=jnp.float32,
        )

    blk = (
        jax.lax.dot_general(
            adj_ref[...],
            support_ref[...],
            (((1,), (0,)), ((), ())),
            preferred_element_type=jnp.float32,
        )
        + b_ref[...]
    )
    acc_ref[pl.ds(i * _BM, _BM), :] = blk

    @pl.when(i == pl.num_programs(0) - 1)
    def _():
        out_ref[...] = acc_ref[...].T


def kernel(input, adj, weight, bias):
    n, k = input.shape
    m = adj.shape[0]
    f = weight.shape[1]

    out_t = pl.pallas_call(
        _gcn_body,
        grid=(m // _BM,),
        in_specs=[
            pl.BlockSpec((n, k), lambda i: (0, 0)),
            pl.BlockSpec((_BM, n), lambda i: (i, 0)),
            pl.BlockSpec((f, k), lambda i: (0, 0)),
            pl.BlockSpec((1, f), lambda i: (0, 0)),
        ],
        out_specs=pl.BlockSpec((f, m), lambda i: (0, 0)),
        out_shape=jax.ShapeDtypeStruct((f, m), jnp.float32),
        scratch_shapes=[
            pltpu.VMEM((n, f), jnp.float32),
            pltpu.VMEM((m, f), jnp.float32),
        ],
    )(input, adj, weight.T, bias.reshape(1, f))
    return out_t.T


# speedup vs baseline: 1.3779x; 1.3779x over previous
import jax
import jax.numpy as jnp
from jax.experimental import pallas as pl
from jax.experimental.pallas import tpu as pltpu

_BM = 200
_NBUF = 4


def _gcn_body(x_ref, adj_hbm, wt_ref, b_ref, out_ref, buf_ref, support_ref,
              acc_ref, sems):
    nblk = adj_hbm.shape[0] // _BM

    def _copy(blk, slot):
        return pltpu.make_async_copy(
            adj_hbm.at[pl.ds(blk * _BM, _BM), :],
            buf_ref.at[slot],
            sems.at[slot],
        )

    for w in range(_NBUF):
        _copy(w, w).start()

    support_ref[...] = jax.lax.dot_general(
        x_ref[...],
        wt_ref[...],
        (((1,), (1,)), ((), ())),
        preferred_element_type=jnp.float32,
    )

    for i in range(nblk):
        slot = i % _NBUF
        _copy(i, slot).wait()
        blk = (
            jax.lax.dot_general(
                buf_ref[slot],
                support_ref[...],
                (((1,), (0,)), ((), ())),
                preferred_element_type=jnp.float32,
            )
            + b_ref[...]
        )
        acc_ref[i * _BM:(i + 1) * _BM, :] = blk
        if i + _NBUF < nblk:
            _copy(i + _NBUF, slot).start()

    out_ref[...] = acc_ref[...].T


def kernel(input, adj, weight, bias):
    n, k = input.shape
    m = adj.shape[0]
    f = weight.shape[1]

    out_t = pl.pallas_call(
        _gcn_body,
        in_specs=[
            pl.BlockSpec((n, k), lambda: (0, 0)),
            pl.BlockSpec(memory_space=pl.ANY),
            pl.BlockSpec((f, k), lambda: (0, 0)),
            pl.BlockSpec((1, f), lambda: (0, 0)),
        ],
        out_specs=pl.BlockSpec((f, m), lambda: (0, 0)),
        out_shape=jax.ShapeDtypeStruct((f, m), jnp.float32),
        scratch_shapes=[
            pltpu.VMEM((_NBUF, _BM, n), jnp.float32),
            pltpu.VMEM((n, f), jnp.float32),
            pltpu.VMEM((m, f), jnp.float32),
            pltpu.SemaphoreType.DMA((_NBUF,)),
        ],
    )(input, adj, weight.T, bias.reshape(1, f))
    return out_t.T
